# Initial kernel scaffold; baseline (speedup 1.0000x reference)
import jax
import jax.numpy as jnp
from jax.experimental import pallas as pl


def kernel(x, position_ids, cos_cached, sin_cached):
    raise NotImplementedError("write your pallas kernel here")


# speedup vs baseline: 1.6237x; 1.6237x over previous
import functools

import jax
import jax.numpy as jnp
from jax import lax
from jax.experimental import pallas as pl
from jax.experimental.pallas import tpu as pltpu
from jax.experimental.pallas import tpu_sc as plsc

HEAD_DIM = 256
B_TOTAL = 32768
NUM_CORES = 2
NUM_SUBCORES = 16
NUM_WORKERS = NUM_CORES * NUM_SUBCORES
B_PER_W = B_TOTAL // NUM_WORKERS
CHUNK = 128
N_CHUNKS = B_PER_W // CHUNK

_mesh = plsc.VectorSubcoreMesh(core_axis_name="c", subcore_axis_name="s")


@functools.partial(
    pl.kernel,
    mesh=_mesh,
    out_type=[
        jax.ShapeDtypeStruct((B_TOTAL, HEAD_DIM), jnp.float32),
        jax.ShapeDtypeStruct((B_TOTAL, HEAD_DIM), jnp.float32),
    ],
    scratch_types=[
        pltpu.VMEM((B_PER_W,), jnp.int32),
        pltpu.VMEM((CHUNK, HEAD_DIM), jnp.float32),
        pltpu.VMEM((CHUNK, HEAD_DIM), jnp.float32),
        pltpu.SemaphoreType.DMA,
    ],
)
def _rope_gather(cos_hbm, sin_hbm, idx_hbm, out_cos, out_sin,
                 idx_v, cos_v, sin_v, sem):
    wid = lax.axis_index("s") * NUM_CORES + lax.axis_index("c")
    base = wid * B_PER_W
    pltpu.sync_copy(idx_hbm.at[pl.ds(base, B_PER_W)], idx_v)
    for ci in range(N_CHUNKS):
        off = ci * CHUNK
        idx_c = idx_v.at[pl.ds(off, CHUNK)]
        g_cos = pltpu.async_copy(cos_hbm.at[idx_c], cos_v, sem)
        g_sin = pltpu.async_copy(sin_hbm.at[idx_c], sin_v, sem)
        g_cos.wait()
        g_sin.wait()
        pltpu.sync_copy(cos_v, out_cos.at[pl.ds(base + off, CHUNK)])
        pltpu.sync_copy(sin_v, out_sin.at[pl.ds(base + off, CHUNK)])


def kernel(x, position_ids, cos_cached, sin_cached):
    idx = position_ids.reshape(-1)
    cos, sin = _rope_gather(cos_cached, sin_cached, idx)
    out_shape = (*position_ids.shape, cos_cached.shape[-1])
    return (cos.reshape(out_shape).astype(x.dtype),
            sin.reshape(out_shape).astype(x.dtype))
